# Initial kernel scaffold; baseline (speedup 1.0000x reference)
import jax
import jax.numpy as jnp
from jax.experimental import pallas as pl


def kernel(x, W1, b1, g1, be1, W2, b2, g2, be2, Wt, bt, gt, bet, Wd, bd, gd, bed, Wo, bo):
    raise NotImplementedError("write your pallas kernel here")


# speedup vs baseline: 6.6773x; 6.6773x over previous
import jax
import jax.numpy as jnp
from jax.experimental import pallas as pl
from jax.experimental.pallas import tpu as pltpu

_EPS = 1e-5


def _predict_kernel(bet_ref, wd_ref, bd_ref, gd_ref, bed_ref,
                    wot_ref, bo_ref, o_ref):
    _, P, N, F = o_ref.shape

    c = jnp.maximum(bet_ref[0], 0.0)

    v = [c * wd_ref[p] + bd_ref[p] for p in range(P)]
    mean = v[0]
    for p in range(1, P):
        mean = mean + v[p]
    mean = mean * (1.0 / P)
    var = jnp.square(v[0] - mean)
    for p in range(1, P):
        var = var + jnp.square(v[p] - mean)
    var = var * (1.0 / P)
    inv = jax.lax.rsqrt(var + _EPS)

    s = jnp.sum(wot_ref[...], axis=0, keepdims=True)

    for p in range(P):
        u_p = (v[p] - mean) * inv * gd_ref[p] + bed_ref[p]
        row = u_p * s + bo_ref[...]
        o_ref[0, p] = jnp.broadcast_to(row, (N, F))


def kernel(x, W1, b1, g1, be1, W2, b2, g2, be2,
           Wt, bt, gt, bet, Wd, bd, gd, bed, Wo, bo):
    B, T, N, F = x.shape
    P = Wd.shape[0]
    M = Wo.shape[1]

    smem = pl.BlockSpec(memory_space=pltpu.MemorySpace.SMEM)
    rep2 = lambda i: (0, 0)

    out = pl.pallas_call(
        _predict_kernel,
        out_shape=jax.ShapeDtypeStruct((B, P, N, F), jnp.float32),
        grid_spec=pltpu.PrefetchScalarGridSpec(
            num_scalar_prefetch=0,
            grid=(B,),
            in_specs=[
                smem,
                smem,
                smem,
                smem,
                smem,
                pl.BlockSpec((M, F), rep2),
                pl.BlockSpec((1, F), rep2),
            ],
            out_specs=pl.BlockSpec((1, P, N, F), lambda i: (i, 0, 0, 0)),
        ),
        compiler_params=pltpu.CompilerParams(
            dimension_semantics=("parallel",)),
    )(bet, Wd.reshape(P), bd, gd, bed, Wo.T, bo.reshape(1, F))
    return out
